# baseline (speedup 1.0000x reference)
import functools

import jax
import jax.numpy as jnp
from jax import lax
from jax.experimental import pallas as pl
from jax.experimental.pallas import tpu as pltpu
from jax.experimental.pallas import tpu_sc as plsc

_VOCAB = 100000
_WIDTH = 256
_B = 4096
_L = 20
_STEPS = 4
_IN_DIM = 3 * _WIDTH
_N_DCN = 4
_SCALE = (_STEPS ** -0.5) * (_L ** -0.5)

_NC = 2
_NS = 16
_NW = _NC * _NS
_ROWS_W = _B // _NW
_IDS_W = _ROWS_W * _L
_K = _STEPS * _L
_LANES = 16


def _mueller(x):
    x = (x >> 16 ^ x) * jnp.int32(73244475)
    x = (x >> 16 ^ x) * jnp.int32(73244475)
    x = x >> 16 ^ x
    return x


def _make_emb_kernel(n_rows, base_row):
    mesh = plsc.VectorSubcoreMesh(core_axis_name="c", subcore_axis_name="s")

    rows_w = n_rows // _NW
    ids_w = rows_w * _L
    n_pairs = rows_w // 2
    ids_off = base_row * _L

    @functools.partial(
        pl.kernel,
        mesh=mesh,
        out_type=jax.ShapeDtypeStruct((n_rows, _WIDTH), jnp.float32),
        scratch_types=[
            pltpu.VMEM((ids_w,), jnp.int32),
            pltpu.VMEM((_STEPS * ids_w,), jnp.int32),
            pltpu.VMEM((2, _STEPS, 2 * _L, 128), jnp.int32),
            pltpu.VMEM((rows_w, _WIDTH), jnp.float32),
            pltpu.SemaphoreType.DMA,
            pltpu.SemaphoreType.DMA,
        ],
    )
    def emb(ids_hbm, table_hbm, out_hbm, ids_v, idx_v, gbuf, acc, sem0, sem1):
        wid = lax.axis_index("s") * _NC + lax.axis_index("c")
        ids_base = ids_off + wid * ids_w

        pltpu.sync_copy(ids_hbm.at[pl.ds(ids_base, ids_w)], ids_v)

        def hash_body(i, _):
            cur = ids_v[pl.ds(i * _LANES, _LANES)]
            for s in range(_STEPS):
                cur = _mueller(cur)
                if s % 2 == 1:
                    cur = cur ^ jnp.int32(-1)
                r = lax.rem(cur, jnp.int32(_VOCAB))
                r = jnp.where(r < 0, r + jnp.int32(_VOCAB), r)
                idx_v[pl.ds(s * ids_w + i * _LANES, _LANES)] = r
            return 0

        lax.fori_loop(0, ids_w // _LANES, hash_body, 0)

        sems = (sem0, sem1)

        def start(rp, slot):
            for s in range(_STEPS):
                pltpu.async_copy(
                    table_hbm.at[idx_v.at[pl.ds(s * ids_w + rp * 2 * _L,
                                                2 * _L)]],
                    gbuf.at[slot, s],
                    sems[slot],
                )

        def wait(rp, slot):
            for s in range(_STEPS):
                pltpu.make_async_copy(
                    table_hbm.at[idx_v.at[pl.ds(s * ids_w + rp * 2 * _L,
                                                2 * _L)]],
                    gbuf.at[slot, s],
                    sems[slot],
                ).wait()

        def accum(rp, slot):
            for half in range(2):
                carry = tuple(jnp.zeros((_LANES,), jnp.float32)
                              for _ in range(_WIDTH // _LANES))
                for s in range(_STEPS):
                    def jbody(j, c, _s=s, _h=half):
                        new = list(c)
                        for g in range(8):
                            v = gbuf[slot, _s, _h * _L + j,
                                     pl.ds(g * _LANES, _LANES)]
                            a = lax.bitcast_convert_type(v << 16,
                                                         jnp.float32)
                            b = lax.bitcast_convert_type(
                                v & jnp.int32(-65536), jnp.float32)
                            new[2 * g] = new[2 * g] + a
                            new[2 * g + 1] = new[2 * g + 1] + b
                        return tuple(new)
                    carry = lax.fori_loop(0, _L, jbody, carry, unroll=2)
                for k in range(_WIDTH // _LANES):
                    acc[rp * 2 + half, pl.ds(k * _LANES, _LANES)] = (
                        carry[k] * _SCALE)

        start(0, 0)
        start(1, 1)

        def loop_body(it, _):
            for slot in range(2):
                rp = it * 2 + slot
                wait(rp, slot)
                accum(rp, slot)

                @pl.when(rp + 2 < n_pairs)
                def _():
                    start(rp + 2, slot)
            return 0

        lax.fori_loop(0, n_pairs // 2, loop_body, 0)

        pltpu.sync_copy(acc, out_hbm.at[pl.ds(wid * rows_w, rows_w)])

    return emb


_CHUNKS = ((0, 2048), (2048, 2048))
_emb_lookups = [_make_emb_kernel(n, b) for b, n in _CHUNKS]


_BM = 512


def _dense_body(z_ref, rat_ref, age_ref, rt_ref, tw_ref, tb_ref,
                u_ref, ub_ref, v_ref, vb_ref,
                w0_ref, b0_ref, w1_ref, b1_ref, w2_ref, b2_ref, w3_ref, b3_ref,
                out_ref):
    z = z_ref[...]
    age = age_ref[...]
    t = jnp.sin(age * tw_ref[...] + tb_ref[...])
    rat = rat_ref[...]
    r = jnp.where(rat == 0, rt_ref[0:1, :],
                  jnp.where(rat == 1, rt_ref[1:2, :], rt_ref[2:3, :]))
    x = jnp.concatenate([z, t, r], axis=-1)
    y = x
    for i in range(_N_DCN):
        u = jnp.dot(y, u_ref[i], preferred_element_type=jnp.float32) + ub_ref[i]
        v = jnp.dot(u, v_ref[i], preferred_element_type=jnp.float32) + vb_ref[i]
        y = y + x * v
    h = x + y
    for w, b, last in ((w0_ref, b0_ref, False), (w1_ref, b1_ref, False),
                       (w2_ref, b2_ref, False), (w3_ref, b3_ref, True)):
        h = jnp.dot(h.astype(jnp.bfloat16), w[...].astype(jnp.bfloat16),
                    preferred_element_type=jnp.float32) + b[...]
        if not last:
            h = jax.nn.gelu(h)
    out_ref[...] = h


def _full(shape):
    nd = len(shape)
    return pl.BlockSpec(shape, lambda i, _n=nd: (0,) * _n)


def _dense_call(base_row, z, rat2, age2, rating_table, time_W, time_b2,
                dcn_U, dcn_Ub3, dcn_V, dcn_Vb3,
                w0, b02, w1, b12, w2, b22, w3, b32):
    n_rows = z.shape[0]
    bm = _BM if (n_rows % _BM == 0 and base_row % _BM == 0) else 256
    grid = (n_rows // bm,)
    off = base_row // bm
    mlp0, mlp1, mlp2 = w0.shape[1], w1.shape[1], w2.shape[1]
    return pl.pallas_call(
        _dense_body,
        grid=grid,
        in_specs=[
            pl.BlockSpec((bm, _WIDTH), lambda i: (i, 0)),
            pl.BlockSpec((bm, 1), lambda i: (i + off, 0)),
            pl.BlockSpec((bm, 1), lambda i: (i + off, 0)),
            _full((3, _WIDTH)),
            _full((1, _WIDTH)),
            _full((1, _WIDTH)),
            _full((_N_DCN, _IN_DIM, 64)),
            _full((_N_DCN, 1, 64)),
            _full((_N_DCN, 64, _IN_DIM)),
            _full((_N_DCN, 1, _IN_DIM)),
            _full((_IN_DIM, mlp0)),
            _full((1, mlp0)),
            _full((mlp0, mlp1)),
            _full((1, mlp1)),
            _full((mlp1, mlp2)),
            _full((1, mlp2)),
            _full((mlp2, 1)),
            _full((1, 1)),
        ],
        out_specs=pl.BlockSpec((bm, 1), lambda i: (i, 0)),
        out_shape=jax.ShapeDtypeStruct((n_rows, 1), jnp.float32),
        compiler_params=pltpu.CompilerParams(
            dimension_semantics=("arbitrary",),
        ),
    )(z, rat2, age2, rating_table, time_W, time_b2,
      dcn_U, dcn_Ub3, dcn_V, dcn_Vb3,
      w0, b02, w1, b12, w2, b22, w3, b32)


def kernel(id_tag, rating, age, embed_table, rating_table, time_W, time_b,
           dcn_U, dcn_Ub, dcn_V, dcn_Vb,
           mlp_W0, mlp_b0, mlp_W1, mlp_b1, mlp_W2, mlp_b2, mlp_W3, mlp_b3):
    ids_flat = id_tag.reshape(-1)
    rat2 = rating.reshape(_B, 1)
    age2 = age.reshape(_B, 1)
    tbl16 = (embed_table.astype(jnp.bfloat16)
             .reshape(_VOCAB, 8, 2, 16)
             .swapaxes(2, 3))
    tbl32 = lax.bitcast_convert_type(tbl16, jnp.int32).reshape(_VOCAB, 128)
    zs = [emb(ids_flat, tbl32) for emb in _emb_lookups]

    def dense(c, z):
        return _dense_call(
            _CHUNKS[c][0],
            z,
            rat2,
            age2,
            rating_table,
            time_W,
            time_b.reshape(1, _WIDTH),
            dcn_U,
            dcn_Ub.reshape(_N_DCN, 1, 64),
            dcn_V,
            dcn_Vb.reshape(_N_DCN, 1, _IN_DIM),
            mlp_W0, mlp_b0.reshape(1, -1),
            mlp_W1, mlp_b1.reshape(1, -1),
            mlp_W2, mlp_b2.reshape(1, -1),
            mlp_W3, mlp_b3.reshape(1, -1),
        )

    outs = [dense(c, zs[c]) for c in range(len(_CHUNKS))]
    order = sorted(range(len(_CHUNKS)), key=lambda c: _CHUNKS[c][0])
    return jnp.concatenate([outs[c] for c in order], axis=0)[:, 0]


# speedup vs baseline: 1.8065x; 1.8065x over previous
import functools

import jax
import jax.numpy as jnp
from jax import lax
from jax.experimental import pallas as pl
from jax.experimental.pallas import tpu as pltpu
from jax.experimental.pallas import tpu_sc as plsc

_VOCAB = 100000
_WIDTH = 256
_B = 4096
_L = 20
_STEPS = 4
_IN_DIM = 3 * _WIDTH
_N_DCN = 4
_SCALE = (_STEPS ** -0.5) * (_L ** -0.5)

_NC = 2
_NS = 16
_NW = _NC * _NS
_ROWS_W = _B // _NW
_IDS_W = _ROWS_W * _L
_K = _STEPS * _L
_LANES = 16


def _mueller(x):
    x = (x >> 16 ^ x) * jnp.int32(73244475)
    x = (x >> 16 ^ x) * jnp.int32(73244475)
    x = x >> 16 ^ x
    return x


def _make_emb_kernel(n_rows, base_row):
    mesh = plsc.VectorSubcoreMesh(core_axis_name="c", subcore_axis_name="s")

    rows_w = n_rows // _NW
    ids_w = rows_w * _L
    n_pairs = rows_w // 2
    ids_off = base_row * _L

    @functools.partial(
        pl.kernel,
        mesh=mesh,
        out_type=jax.ShapeDtypeStruct((n_rows, _WIDTH), jnp.float32),
        scratch_types=[
            pltpu.VMEM((ids_w,), jnp.int32),
            pltpu.VMEM((_STEPS * ids_w,), jnp.int32),
            pltpu.VMEM((2, _STEPS, 2 * _L, _WIDTH), jnp.float32),
            pltpu.VMEM((rows_w, _WIDTH), jnp.float32),
            pltpu.SemaphoreType.DMA,
            pltpu.SemaphoreType.DMA,
        ],
    )
    def emb(ids_hbm, table_hbm, out_hbm, ids_v, idx_v, gbuf, acc, sem0, sem1):
        wid = lax.axis_index("s") * _NC + lax.axis_index("c")
        ids_base = ids_off + wid * ids_w

        pltpu.sync_copy(ids_hbm.at[pl.ds(ids_base, ids_w)], ids_v)

        def hash_body(i, _):
            cur = ids_v[pl.ds(i * _LANES, _LANES)]
            for s in range(_STEPS):
                cur = _mueller(cur)
                if s % 2 == 1:
                    cur = cur ^ jnp.int32(-1)
                r = lax.rem(cur, jnp.int32(_VOCAB))
                r = jnp.where(r < 0, r + jnp.int32(_VOCAB), r)
                idx_v[pl.ds(s * ids_w + i * _LANES, _LANES)] = r
            return 0

        lax.fori_loop(0, ids_w // _LANES, hash_body, 0)

        sems = (sem0, sem1)

        def start(rp, slot):
            for s in range(_STEPS):
                pltpu.async_copy(
                    table_hbm.at[idx_v.at[pl.ds(s * ids_w + rp * 2 * _L,
                                                2 * _L)]],
                    gbuf.at[slot, s],
                    sems[slot],
                )

        def wait(rp, slot):
            for s in range(_STEPS):
                pltpu.make_async_copy(
                    table_hbm.at[idx_v.at[pl.ds(s * ids_w + rp * 2 * _L,
                                                2 * _L)]],
                    gbuf.at[slot, s],
                    sems[slot],
                ).wait()

        def accum(rp, slot):
            for half in range(2):
                carry = tuple(jnp.zeros((_LANES,), jnp.float32)
                              for _ in range(_WIDTH // _LANES))
                for s in range(_STEPS):
                    def jbody(j, c, _s=s, _h=half):
                        return tuple(
                            c[k] + gbuf[slot, _s, _h * _L + j,
                                        pl.ds(k * _LANES, _LANES)]
                            for k in range(_WIDTH // _LANES)
                        )
                    carry = lax.fori_loop(0, _L, jbody, carry, unroll=2)
                for k in range(_WIDTH // _LANES):
                    acc[rp * 2 + half, pl.ds(k * _LANES, _LANES)] = (
                        carry[k] * _SCALE)

        start(0, 0)
        start(1, 1)

        def loop_body(it, _):
            for slot in range(2):
                rp = it * 2 + slot
                wait(rp, slot)
                accum(rp, slot)

                @pl.when(rp + 2 < n_pairs)
                def _():
                    start(rp + 2, slot)
            return 0

        lax.fori_loop(0, n_pairs // 2, loop_body, 0)

        pltpu.sync_copy(acc, out_hbm.at[pl.ds(wid * rows_w, rows_w)])

    return emb


_CHUNKS = ((0, 2048), (2048, 2048))
_emb_lookups = [_make_emb_kernel(n, b) for b, n in _CHUNKS]


_BM = 512


def _dense_body(z_ref, rat_ref, age_ref, rt_ref, tw_ref, tb_ref,
                u_ref, ub_ref, v_ref, vb_ref,
                w0_ref, b0_ref, w1_ref, b1_ref, w2_ref, b2_ref, w3_ref, b3_ref,
                out_ref):
    z = z_ref[...]
    age = age_ref[...]
    t = jnp.sin(age * tw_ref[...] + tb_ref[...])
    rat = rat_ref[...]
    r = jnp.where(rat == 0, rt_ref[0:1, :],
                  jnp.where(rat == 1, rt_ref[1:2, :], rt_ref[2:3, :]))
    x = jnp.concatenate([z, t, r], axis=-1)
    y = x
    for i in range(_N_DCN):
        u = jnp.dot(y, u_ref[i], preferred_element_type=jnp.float32) + ub_ref[i]
        v = jnp.dot(u, v_ref[i], preferred_element_type=jnp.float32) + vb_ref[i]
        y = y + x * v
    h = x + y
    for w, b, last in ((w0_ref, b0_ref, False), (w1_ref, b1_ref, False),
                       (w2_ref, b2_ref, False), (w3_ref, b3_ref, True)):
        h = jnp.dot(h.astype(jnp.bfloat16), w[...].astype(jnp.bfloat16),
                    preferred_element_type=jnp.float32) + b[...]
        if not last:
            h = jax.nn.gelu(h)
    out_ref[...] = h


def _full(shape):
    nd = len(shape)
    return pl.BlockSpec(shape, lambda i, _n=nd: (0,) * _n)


def _dense_call(base_row, z, rat2, age2, rating_table, time_W, time_b2,
                dcn_U, dcn_Ub3, dcn_V, dcn_Vb3,
                w0, b02, w1, b12, w2, b22, w3, b32):
    n_rows = z.shape[0]
    bm = _BM if (n_rows % _BM == 0 and base_row % _BM == 0) else 256
    grid = (n_rows // bm,)
    off = base_row // bm
    mlp0, mlp1, mlp2 = w0.shape[1], w1.shape[1], w2.shape[1]
    return pl.pallas_call(
        _dense_body,
        grid=grid,
        in_specs=[
            pl.BlockSpec((bm, _WIDTH), lambda i: (i, 0)),
            pl.BlockSpec((bm, 1), lambda i: (i + off, 0)),
            pl.BlockSpec((bm, 1), lambda i: (i + off, 0)),
            _full((3, _WIDTH)),
            _full((1, _WIDTH)),
            _full((1, _WIDTH)),
            _full((_N_DCN, _IN_DIM, 64)),
            _full((_N_DCN, 1, 64)),
            _full((_N_DCN, 64, _IN_DIM)),
            _full((_N_DCN, 1, _IN_DIM)),
            _full((_IN_DIM, mlp0)),
            _full((1, mlp0)),
            _full((mlp0, mlp1)),
            _full((1, mlp1)),
            _full((mlp1, mlp2)),
            _full((1, mlp2)),
            _full((mlp2, 1)),
            _full((1, 1)),
        ],
        out_specs=pl.BlockSpec((bm, 1), lambda i: (i, 0)),
        out_shape=jax.ShapeDtypeStruct((n_rows, 1), jnp.float32),
        compiler_params=pltpu.CompilerParams(
            dimension_semantics=("arbitrary",),
        ),
    )(z, rat2, age2, rating_table, time_W, time_b2,
      dcn_U, dcn_Ub3, dcn_V, dcn_Vb3,
      w0, b02, w1, b12, w2, b22, w3, b32)


def kernel(id_tag, rating, age, embed_table, rating_table, time_W, time_b,
           dcn_U, dcn_Ub, dcn_V, dcn_Vb,
           mlp_W0, mlp_b0, mlp_W1, mlp_b1, mlp_W2, mlp_b2, mlp_W3, mlp_b3):
    ids_flat = id_tag.reshape(-1)
    rat2 = rating.reshape(_B, 1)
    age2 = age.reshape(_B, 1)
    zs = [emb(ids_flat, embed_table) for emb in _emb_lookups]

    def dense(c, z):
        return _dense_call(
            _CHUNKS[c][0],
            z,
            rat2,
            age2,
            rating_table,
            time_W,
            time_b.reshape(1, _WIDTH),
            dcn_U,
            dcn_Ub.reshape(_N_DCN, 1, 64),
            dcn_V,
            dcn_Vb.reshape(_N_DCN, 1, _IN_DIM),
            mlp_W0, mlp_b0.reshape(1, -1),
            mlp_W1, mlp_b1.reshape(1, -1),
            mlp_W2, mlp_b2.reshape(1, -1),
            mlp_W3, mlp_b3.reshape(1, -1),
        )

    outs = [dense(c, zs[c]) for c in range(len(_CHUNKS))]
    order = sorted(range(len(_CHUNKS)), key=lambda c: _CHUNKS[c][0])
    return jnp.concatenate([outs[c] for c in order], axis=0)[:, 0]
